# Initial kernel scaffold; baseline (speedup 1.0000x reference)
import jax
import jax.numpy as jnp
from jax.experimental import pallas as pl


def kernel(tokens, table):
    raise NotImplementedError("write your pallas kernel here")


# speedup vs baseline: 8.9331x; 8.9331x over previous
import functools
import math

import jax
import jax.numpy as jnp
from jax import lax
from jax.experimental import pallas as pl
from jax.experimental.pallas import tpu as pltpu
from jax.experimental.pallas import tpu_sc as plsc

VOCAB = 100000
DIM = 64
SEQ = 1500
BATCH = 1024
SEQ_PAD = 1504

NC = 2
NS = 16
NW = NC * NS
ROWS_PER_W = BATCH // NW

CHUNKS = ((0, 512), (512, 512), (1024, 476))
CLMAX = 512
IGS = 128
ADD_UNROLL = 4


def _pe_table():
    position = jnp.arange(SEQ, dtype=jnp.float32)[:, None]
    div_term = jnp.exp(
        jnp.arange(0, DIM, 2, dtype=jnp.float32) * (-(math.log(10000.0) / DIM))
    )
    ang = position * div_term
    pe = jnp.zeros((SEQ, DIM), dtype=jnp.float32)
    pe = pe.at[:, 0::2].set(jnp.sin(ang))
    pe = pe.at[:, 1::2].set(jnp.cos(ang))
    return pe


@functools.partial(
    pl.kernel,
    mesh=plsc.VectorSubcoreMesh(core_axis_name="c", subcore_axis_name="s"),
    out_type=jax.ShapeDtypeStruct((BATCH, SEQ, DIM), jnp.float32),
    scratch_types=[
        pltpu.VMEM((CLMAX // IGS, IGS), jnp.int32),
        pltpu.VMEM((CLMAX, DIM), jnp.float32),
        pltpu.VMEM((CLMAX, DIM), jnp.float32),
        pltpu.SemaphoreType.DMA,
    ],
    compiler_params=pltpu.CompilerParams(use_tc_tiling_on_sc=False),
)
def _seq_encode(tok_hbm, pe_hbm, table_hbm, out_hbm, idx_v, pe_v, rows_v, sem):
    wid = lax.axis_index("s") * NC + lax.axis_index("c")
    for off, cl in CHUNKS:
        n_full, rem = divmod(cl, IGS)
        rem_pad = -(-rem // 8) * 8
        pltpu.sync_copy(pe_hbm.at[pl.ds(off, cl), :], pe_v.at[pl.ds(0, cl), :])

        def row_body(r, carry, off=off, cl=cl, n_full=n_full, rem=rem_pad):
            g = wid * ROWS_PER_W + r
            tok_base = g * SEQ_PAD + off
            for j in range(n_full):
                pltpu.sync_copy(
                    tok_hbm.at[pl.ds(tok_base + j * IGS, IGS)], idx_v.at[j]
                )
            if rem:
                pltpu.sync_copy(
                    tok_hbm.at[pl.ds(tok_base + n_full * IGS, rem)],
                    idx_v.at[n_full, pl.ds(0, rem)],
                )
            handles = []
            for j in range(n_full):
                handles.append(
                    pltpu.async_copy(
                        table_hbm.at[idx_v.at[j]],
                        rows_v.at[pl.ds(j * IGS, IGS), :],
                        sem,
                    )
                )
            if rem:
                handles.append(
                    pltpu.async_copy(
                        table_hbm.at[idx_v.at[n_full, pl.ds(0, rem)]],
                        rows_v.at[pl.ds(n_full * IGS, rem), :],
                        sem,
                    )
                )
            for h in handles:
                h.wait()

            def add_body(i, c):
                for u in range(ADD_UNROLL):
                    p = i * ADD_UNROLL + u
                    for v in range(DIM // 16):
                        plsc.addupdate(
                            rows_v.at[p, pl.ds(v * 16, 16)],
                            pe_v[p, pl.ds(v * 16, 16)],
                        )
                return c

            lax.fori_loop(0, cl // ADD_UNROLL, add_body, 0)

            pltpu.sync_copy(
                rows_v.at[pl.ds(0, cl), :],
                out_hbm.at[g, pl.ds(off, cl), :],
            )
            return carry

        lax.fori_loop(0, ROWS_PER_W, row_body, 0)


def kernel(tokens, table):
    pe = _pe_table()
    tok_flat = jnp.pad(tokens, ((0, 0), (0, SEQ_PAD - SEQ))).reshape(-1)
    return _seq_encode(tok_flat, pe, table)


# speedup vs baseline: 9.9813x; 1.1173x over previous
import functools
import math

import jax
import jax.numpy as jnp
from jax import lax
from jax.experimental import pallas as pl
from jax.experimental.pallas import tpu as pltpu
from jax.experimental.pallas import tpu_sc as plsc

VOCAB = 100000
DIM = 64
SEQ = 1500
BATCH = 1024
SEQ_PAD = 1504

NC = 2
NS = 16
NW = NC * NS
ROWS_PER_W = BATCH // NW

CHUNKS = ((0, 512), (512, 512), (1024, 476))
CLMAX = 512
IGS = 128


def _pe_table():
    position = jnp.arange(SEQ, dtype=jnp.float32)[:, None]
    div_term = jnp.exp(
        jnp.arange(0, DIM, 2, dtype=jnp.float32) * (-(math.log(10000.0) / DIM))
    )
    ang = position * div_term
    pe = jnp.zeros((SEQ, DIM), dtype=jnp.float32)
    pe = pe.at[:, 0::2].set(jnp.sin(ang))
    pe = pe.at[:, 1::2].set(jnp.cos(ang))
    return pe


@functools.partial(
    pl.kernel,
    mesh=plsc.VectorSubcoreMesh(core_axis_name="c", subcore_axis_name="s"),
    out_type=jax.ShapeDtypeStruct((BATCH, SEQ, DIM), jnp.float32),
    scratch_types=[
        pltpu.VMEM((CLMAX, DIM), jnp.float32),
        pltpu.VMEM((CLMAX,), jnp.int32),
        pltpu.VMEM((CLMAX,), jnp.int32),
        pltpu.VMEM((CLMAX, DIM), jnp.float32),
        pltpu.VMEM((CLMAX, DIM), jnp.float32),
        pltpu.SemaphoreType.DMA,
        pltpu.SemaphoreType.DMA,
        pltpu.SemaphoreType.DMA,
        pltpu.SemaphoreType.DMA,
        pltpu.SemaphoreType.DMA,
        pltpu.SemaphoreType.DMA,
    ],
    compiler_params=pltpu.CompilerParams(use_tc_tiling_on_sc=False),
)
def _seq_encode(tok_hbm, pe_hbm, table_hbm, out_hbm,
                pe_v, idx0, idx1, rows0, rows1,
                sem_i0, sem_i1, sem_g0, sem_g1, sem_s0, sem_s1):
    wid = lax.axis_index("s") * NC + lax.axis_index("c")

    for off, cl in CHUNKS:
        cl_pad = -(-cl // 8) * 8
        n_g, rem = divmod(cl_pad, IGS)
        unroll = 8 if cl % 8 == 0 else 4

        pltpu.sync_copy(pe_hbm.at[pl.ds(off, cl), :], pe_v.at[pl.ds(0, cl), :])

        def fire_gathers(idx_v, rows_v, sem, n_g=n_g, rem=rem):
            handles = []
            for j in range(n_g):
                handles.append(pltpu.async_copy(
                    table_hbm.at[idx_v.at[pl.ds(j * IGS, IGS)]],
                    rows_v.at[pl.ds(j * IGS, IGS), :], sem))
            if rem:
                handles.append(pltpu.async_copy(
                    table_hbm.at[idx_v.at[pl.ds(n_g * IGS, rem)]],
                    rows_v.at[pl.ds(n_g * IGS, rem), :], sem))
            return handles

        def add_pe(rows_v, cl=cl, unroll=unroll):
            def add_body(i, c):
                for u in range(unroll):
                    p = i * unroll + u
                    for v in range(DIM // 16):
                        plsc.addupdate(
                            rows_v.at[p, pl.ds(v * 16, 16)],
                            pe_v[p, pl.ds(v * 16, 16)],
                        )
                return c
            lax.fori_loop(0, cl // unroll, add_body, 0)

        def drain_store(rows_v, sem, off=off, cl=cl):
            pltpu.make_async_copy(
                rows_v.at[pl.ds(0, cl), :], out_hbm.at[0, pl.ds(off, cl), :], sem
            ).wait()

        def pair_body(r2, carry, off=off, cl=cl, cl_pad=cl_pad):
            ga = wid * ROWS_PER_W + 2 * r2
            gb = ga + 1

            @pl.when(r2 > 0)
            def _():
                drain_store(rows0, sem_s0)
                drain_store(rows1, sem_s1)

            ha = pltpu.async_copy(
                tok_hbm.at[pl.ds(ga * SEQ_PAD + off, cl_pad)],
                idx0.at[pl.ds(0, cl_pad)], sem_i0)
            hb = pltpu.async_copy(
                tok_hbm.at[pl.ds(gb * SEQ_PAD + off, cl_pad)],
                idx1.at[pl.ds(0, cl_pad)], sem_i1)

            ha.wait()
            hga = fire_gathers(idx0, rows0, sem_g0)
            hb.wait()
            hgb = fire_gathers(idx1, rows1, sem_g1)

            for h in hga:
                h.wait()
            add_pe(rows0)
            pltpu.async_copy(
                rows0.at[pl.ds(0, cl), :], out_hbm.at[ga, pl.ds(off, cl), :], sem_s0)

            for h in hgb:
                h.wait()
            add_pe(rows1)
            pltpu.async_copy(
                rows1.at[pl.ds(0, cl), :], out_hbm.at[gb, pl.ds(off, cl), :], sem_s1)
            return carry

        lax.fori_loop(0, ROWS_PER_W // 2, pair_body, 0)
        drain_store(rows0, sem_s0)
        drain_store(rows1, sem_s1)


def kernel(tokens, table):
    pe = _pe_table()
    tok_flat = jnp.pad(tokens, ((0, 0), (0, SEQ_PAD - SEQ))).reshape(-1)
    return _seq_encode(tok_flat, pe, table)
